# Initial kernel scaffold; baseline (speedup 1.0000x reference)
import jax
import jax.numpy as jnp
from jax.experimental import pallas as pl


def kernel(query, query_token_embeds_weight, weights_weight):
    raise NotImplementedError("write your pallas kernel here")


# speedup vs baseline: 30.5597x; 30.5597x over previous
import dataclasses
import functools

import jax
import jax.numpy as jnp
from jax import lax
from jax.experimental import pallas as pl
from jax.experimental.pallas import tpu as pltpu
from jax.experimental.pallas import tpu_sc as plsc

_D = 128
_LANES = 16


def _expw_body(w_ref, o_ref):
    w = w_ref[...]
    o_ref[...] = jnp.exp(w - jnp.max(w))


def _sc_pool(q1, table, expw, B, L):
    NW = 32
    RPW = B // NW
    C = 40
    NCH = L // C
    mesh = plsc.VectorSubcoreMesh(core_axis_name="c", subcore_axis_name="s")
    cp = pltpu.CompilerParams()
    if "needs_layout_passes" in pltpu.CompilerParams.__dataclass_fields__:
        cp = dataclasses.replace(cp, needs_layout_passes=False)

    @functools.partial(
        pl.kernel,
        out_type=jax.ShapeDtypeStruct((B, _D), jnp.float32),
        mesh=mesh,
        compiler_params=cp,
        scratch_types=[
            pltpu.VMEM((RPW * L,), jnp.int32),
            pltpu.VMEM((L, _D), jnp.float32),
            pltpu.VMEM((L,), jnp.float32),
            pltpu.VMEM((RPW, _D), jnp.float32),
            pltpu.SemaphoreType.DMA,
            pltpu.SemaphoreType.DMA,
        ],
    )
    def run(q_hbm, t_hbm, ew_hbm, o_hbm, idx_v, rows_v, w_v, out_v, sem_e, sem_w):
        wid = lax.axis_index("s") * 2 + lax.axis_index("c")
        base = wid * RPW
        pltpu.sync_copy(q_hbm.at[pl.ds(base * L, RPW * L)], idx_v)

        @pl.loop(0, RPW)
        def _row(r):
            copies = []
            for h in range(NCH):
                idx_h = idx_v.at[pl.ds(r * L + h * C, C)]
                copies.append(pltpu.async_copy(
                    t_hbm.at[idx_h], rows_v.at[pl.ds(h * C, C)], sem_e))
                copies.append(pltpu.async_copy(
                    ew_hbm.at[idx_h], w_v.at[pl.ds(h * C, C)], sem_w))
            for cp in copies:
                cp.wait()

            zero = jnp.zeros((_LANES,), jnp.float32)

            def body(l, accs):
                sidx = jnp.full((_LANES,), l, dtype=jnp.int32)
                s = plsc.load_gather(w_v, [sidx])
                new = []
                for d in range(_D // _LANES):
                    e = rows_v[l, pl.ds(d * _LANES, _LANES)]
                    new.append(accs[d] + s * e)
                new.append(accs[_D // _LANES] + s)
                return tuple(new)

            accs = lax.fori_loop(0, L, body, tuple([zero] * (_D // _LANES + 1)))
            den = accs[_D // _LANES]
            for d in range(_D // _LANES):
                out_v[r, pl.ds(d * _LANES, _LANES)] = accs[d] / den

        pltpu.sync_copy(out_v, o_hbm.at[pl.ds(base, RPW)])

    return run(q1, table, expw)


def kernel(query, query_token_embeds_weight, weights_weight):
    B, L = query.shape
    V = query_token_embeds_weight.shape[0]
    q1 = query.astype(jnp.int32).reshape(B * L)
    w2d = weights_weight.reshape(V // 125, 125)
    expw2d = pl.pallas_call(
        _expw_body,
        out_shape=jax.ShapeDtypeStruct(w2d.shape, jnp.float32),
    )(w2d)
    expw = expw2d.reshape(V)
    return _sc_pool(q1, query_token_embeds_weight, expw, B, L)
